# Initial kernel scaffold; baseline (speedup 1.0000x reference)
import jax
import jax.numpy as jnp
from jax.experimental import pallas as pl


def kernel(hidden, degree, stack, W, b, edge_index):
    raise NotImplementedError("write your pallas kernel here")


# speedup vs baseline: 11.9956x; 11.9956x over previous
import functools

import jax
import jax.numpy as jnp
from jax import lax
from jax.experimental import pallas as pl
from jax.experimental.pallas import tpu as pltpu
from jax.experimental.pallas import tpu_sc as plsc

N_NODES = 10000
N_EDGES = 320000
D_FEAT = 128

NC = 2
NS = 16
HALF = N_NODES // NC
E_PER_S = N_EDGES // NS
CH = 80
NCH = E_PER_S // CH
N_LOC = 5120
RPT = N_LOC // NS

_sc_mesh = plsc.VectorSubcoreMesh(core_axis_name="c", subcore_axis_name="s")


def _edge_body(hnorm_hbm, srcr_hbm, dstr_hbm, out_hbm,
               src_v, dst_v, rows_a, rows_b, acc, gsem_a, gsem_b):
    cid = lax.axis_index("c")
    sid = lax.axis_index("s")

    zv = jnp.zeros((16,), jnp.float32)

    def zbody(i, _):
        for k in range(D_FEAT // 16):
            rows_a[i, pl.ds(k * 16, 16)] = zv
        return 0

    lax.fori_loop(0, CH, zbody, 0)
    for k in range(RPT // CH):
        pltpu.sync_copy(rows_a, acc.at[pl.ds(sid * RPT + k * CH, CH)])

    pltpu.sync_copy(srcr_hbm.at[sid], src_v)
    pltpu.sync_copy(dstr_hbm.at[cid, sid], dst_v)
    plsc.subcore_barrier()

    pltpu.async_copy(hnorm_hbm.at[src_v.at[0]], rows_a, gsem_a)
    pltpu.async_copy(hnorm_hbm.at[src_v.at[1]], rows_b, gsem_b)

    def body(jj, _):
        ja = 2 * jj
        jb = 2 * jj + 1
        pltpu.make_async_copy(hnorm_hbm.at[src_v.at[ja]], rows_a, gsem_a).wait()
        pltpu.sync_copy(rows_a, acc.at[dst_v.at[ja]], add=True)

        @pl.when(jj < NCH // 2 - 1)
        def _():
            pltpu.async_copy(hnorm_hbm.at[src_v.at[ja + 2]], rows_a, gsem_a)

        pltpu.make_async_copy(hnorm_hbm.at[src_v.at[jb]], rows_b, gsem_b).wait()
        pltpu.sync_copy(rows_b, acc.at[dst_v.at[jb]], add=True)

        @pl.when(jj < NCH // 2 - 1)
        def _():
            pltpu.async_copy(hnorm_hbm.at[src_v.at[jb + 2]], rows_b, gsem_b)

        return 0

    lax.fori_loop(0, NCH // 2, body, 0)

    plsc.subcore_barrier()
    pltpu.sync_copy(acc.at[pl.ds(sid * RPT, RPT)],
                    out_hbm.at[cid, pl.ds(sid * RPT, RPT)])


_edge_call = functools.partial(
    pl.kernel,
    out_type=jax.ShapeDtypeStruct((NC, N_LOC, D_FEAT), jnp.float32),
    mesh=_sc_mesh,
    scratch_types=[
        pltpu.VMEM((NCH, CH), jnp.int32),
        pltpu.VMEM((NCH, CH), jnp.int32),
        pltpu.VMEM((CH, D_FEAT), jnp.float32),
        pltpu.VMEM((CH, D_FEAT), jnp.float32),
        pltpu.VMEM_SHARED((N_LOC, D_FEAT), jnp.float32),
        pltpu.SemaphoreType.DMA,
        pltpu.SemaphoreType.DMA,
    ],
)(_edge_body)


ROWS_BLK = 400
N_BLKS = N_NODES // ROWS_BLK


def _hnorm_body(h_ref, d_ref, o_ref):
    o_ref[...] = h_ref[...] / d_ref[...]


def _hnorm(hidden, degree):
    return pl.pallas_call(
        _hnorm_body,
        out_shape=jax.ShapeDtypeStruct((N_NODES, D_FEAT), jnp.float32),
        grid=(N_BLKS,),
        in_specs=[
            pl.BlockSpec((ROWS_BLK, D_FEAT), lambda i: (i, 0)),
            pl.BlockSpec((ROWS_BLK, 1), lambda i: (i, 0)),
        ],
        out_specs=pl.BlockSpec((ROWS_BLK, D_FEAT), lambda i: (i, 0)),
    )(hidden, degree)


APPLY_BLK = 200
APPLY_PER_CORE = HALF // APPLY_BLK
N_APPLY_BLKS = N_NODES // APPLY_BLK


def _apply_body(stack_ref, p_ref, h_ref, wt_ref, b_ref, o_ref):
    red = p_ref[0] + h_ref[...]
    o_ref[:, :D_FEAT] = stack_ref[...]
    o_ref[:, D_FEAT:] = (
        jnp.dot(red, wt_ref[...], preferred_element_type=jnp.float32)
        + b_ref[...]
    )


def _apply(stack, partials, hidden, wt, b2):
    return pl.pallas_call(
        _apply_body,
        out_shape=jax.ShapeDtypeStruct((N_NODES, 2 * D_FEAT), jnp.float32),
        grid=(N_APPLY_BLKS,),
        in_specs=[
            pl.BlockSpec((APPLY_BLK, D_FEAT), lambda i: (i, 0)),
            pl.BlockSpec(
                (1, APPLY_BLK, D_FEAT),
                lambda i: (i // APPLY_PER_CORE, i % APPLY_PER_CORE, 0),
            ),
            pl.BlockSpec((APPLY_BLK, D_FEAT), lambda i: (i, 0)),
            pl.BlockSpec((D_FEAT, D_FEAT), lambda i: (0, 0)),
            pl.BlockSpec((1, D_FEAT), lambda i: (0, 0)),
        ],
        out_specs=pl.BlockSpec((APPLY_BLK, 2 * D_FEAT), lambda i: (i, 0)),
    )(stack, partials, hidden, wt, b2)


def kernel(hidden, degree, stack, W, b, edge_index):
    hnorm = _hnorm(hidden, degree)
    srcr = edge_index[0].reshape(NS, NCH, CH)
    dst = edge_index[1]
    d0 = jnp.where(dst < HALF, dst, HALF)
    d1 = jnp.where(dst >= HALF, dst - HALF, HALF)
    dstr = jnp.stack([d0, d1]).reshape(NC, NS, NCH, CH)
    partials = _edge_call(hnorm, srcr, dstr)
    return _apply(stack, partials, hidden, W.T, b.reshape(1, D_FEAT))


# speedup vs baseline: 13.3094x; 1.1095x over previous
import functools

import jax
import jax.numpy as jnp
from jax import lax
from jax.experimental import pallas as pl
from jax.experimental.pallas import tpu as pltpu
from jax.experimental.pallas import tpu_sc as plsc

N_NODES = 10000
N_EDGES = 320000
D_FEAT = 128

NC = 2
NS = 16
HALF = N_NODES // NC
E_PER_S = N_EDGES // NS
CH = 80
NCH = E_PER_S // CH
N_LOC = 5120
RPT = N_LOC // NS

_sc_mesh = plsc.VectorSubcoreMesh(core_axis_name="c", subcore_axis_name="s")


def _edge_body(hnorm_hbm, srcr_hbm, dstr_hbm, out_hbm,
               src_v, dst_v, rows_a, rows_b, acc, gsem_a, gsem_b):
    cid = lax.axis_index("c")
    sid = lax.axis_index("s")

    zv = jnp.zeros((16,), jnp.float32)

    def zbody(i, _):
        for k in range(D_FEAT // 16):
            rows_a[i, pl.ds(k * 16, 16)] = zv
        return 0

    lax.fori_loop(0, CH, zbody, 0)
    for k in range(RPT // CH):
        pltpu.sync_copy(rows_a, acc.at[pl.ds(sid * RPT + k * CH, CH)])

    pltpu.sync_copy(srcr_hbm.at[sid], src_v)
    pltpu.sync_copy(dstr_hbm.at[cid, sid], dst_v)
    plsc.subcore_barrier()

    pltpu.async_copy(hnorm_hbm.at[src_v.at[0]], rows_a, gsem_a)
    pltpu.async_copy(hnorm_hbm.at[src_v.at[1]], rows_b, gsem_b)

    def body(jj, _):
        ja = 2 * jj
        jb = 2 * jj + 1
        pltpu.make_async_copy(hnorm_hbm.at[src_v.at[ja]], rows_a, gsem_a).wait()
        pltpu.sync_copy(rows_a, acc.at[dst_v.at[ja]], add=True)

        @pl.when(jj < NCH // 2 - 1)
        def _():
            pltpu.async_copy(hnorm_hbm.at[src_v.at[ja + 2]], rows_a, gsem_a)

        pltpu.make_async_copy(hnorm_hbm.at[src_v.at[jb]], rows_b, gsem_b).wait()
        pltpu.sync_copy(rows_b, acc.at[dst_v.at[jb]], add=True)

        @pl.when(jj < NCH // 2 - 1)
        def _():
            pltpu.async_copy(hnorm_hbm.at[src_v.at[jb + 2]], rows_b, gsem_b)

        return 0

    lax.fori_loop(0, NCH // 2, body, 0)

    plsc.subcore_barrier()
    pltpu.sync_copy(acc.at[pl.ds(sid * RPT, RPT)],
                    out_hbm.at[cid, pl.ds(sid * RPT, RPT)])


_edge_call = functools.partial(
    pl.kernel,
    out_type=jax.ShapeDtypeStruct((NC, N_LOC, D_FEAT), jnp.float32),
    mesh=_sc_mesh,
    scratch_types=[
        pltpu.VMEM((NCH, CH), jnp.int32),
        pltpu.VMEM((NCH, CH), jnp.int32),
        pltpu.VMEM((CH, D_FEAT), jnp.float32),
        pltpu.VMEM((CH, D_FEAT), jnp.float32),
        pltpu.VMEM_SHARED((N_LOC, D_FEAT), jnp.float32),
        pltpu.SemaphoreType.DMA,
        pltpu.SemaphoreType.DMA,
    ],
)(_edge_body)


ROWS_BLK = 400
N_BLKS = N_NODES // ROWS_BLK


def _hnorm_body(h_ref, d_ref, o_ref):
    o_ref[...] = h_ref[...] / d_ref[...]


def _hnorm(hidden, degree):
    return pl.pallas_call(
        _hnorm_body,
        out_shape=jax.ShapeDtypeStruct((N_NODES, D_FEAT), jnp.float32),
        grid=(N_BLKS,),
        in_specs=[
            pl.BlockSpec((ROWS_BLK, D_FEAT), lambda i: (i, 0)),
            pl.BlockSpec((ROWS_BLK, 1), lambda i: (i, 0)),
        ],
        out_specs=pl.BlockSpec((ROWS_BLK, D_FEAT), lambda i: (i, 0)),
    )(hidden, degree)


APPLY_BLK = 200
APPLY_PER_CORE = HALF // APPLY_BLK
N_APPLY_BLKS = N_NODES // APPLY_BLK


def _apply_body(stack_ref, p_ref, h_ref, wt_ref, b_ref, o_ref):
    red = p_ref[0] + h_ref[...]
    o_ref[:, :D_FEAT] = stack_ref[...]
    o_ref[:, D_FEAT:] = (
        jnp.dot(red, wt_ref[...], preferred_element_type=jnp.float32)
        + b_ref[...]
    )


def _apply(stack, partials, hidden, wt, b2):
    return pl.pallas_call(
        _apply_body,
        out_shape=jax.ShapeDtypeStruct((N_NODES, 2 * D_FEAT), jnp.float32),
        grid=(N_APPLY_BLKS,),
        in_specs=[
            pl.BlockSpec((APPLY_BLK, D_FEAT), lambda i: (i, 0)),
            pl.BlockSpec(
                (1, APPLY_BLK, D_FEAT),
                lambda i: (i // APPLY_PER_CORE, i % APPLY_PER_CORE, 0),
            ),
            pl.BlockSpec((APPLY_BLK, D_FEAT), lambda i: (i, 0)),
            pl.BlockSpec((D_FEAT, D_FEAT), lambda i: (0, 0)),
            pl.BlockSpec((1, D_FEAT), lambda i: (0, 0)),
        ],
        out_specs=pl.BlockSpec((APPLY_BLK, 2 * D_FEAT), lambda i: (i, 0)),
    )(stack, partials, hidden, wt, b2)


def kernel(hidden, degree, stack, W, b, edge_index):
    hnorm = _hnorm(hidden, degree)
    srcr = edge_index[0].reshape(NS, NCH, CH)
    dst = edge_index[1]
    trash = HALF + jnp.arange(N_EDGES, dtype=jnp.int32) % (N_LOC - HALF)
    d0 = jnp.where(dst < HALF, dst, trash)
    d1 = jnp.where(dst >= HALF, dst - HALF, trash)
    dstr = jnp.stack([d0, d1]).reshape(NC, NS, NCH, CH)
    partials = _edge_call(hnorm, srcr, dstr)
    return _apply(stack, partials, hidden, W.T, b.reshape(1, D_FEAT))
